# baseline (speedup 1.0000x reference)
import functools

import jax
import jax.numpy as jnp
from jax import lax
from jax.experimental import pallas as pl
from jax.experimental.pallas import tpu as pltpu
from jax.experimental.pallas import tpu_sc as plsc

_LANES = 16
_NUM_CORES = 2
_NUM_SUBCORES = 16
_NW = _NUM_CORES * _NUM_SUBCORES
_SEQ_PER_CHUNK = 4
_NBUF = 4
_LOOKAHEAD = 2
_PAD_ID = 0


def _window_starts(l: int):
  starts = list(range(0, l - _LANES + 1, _LANES))
  if starts[-1] + _LANES < l:
    starts.append(l - _LANES)
  return tuple(starts)


def _make_gather(b: int, l: int, d: int):
  assert b % (_NW * _SEQ_PER_CHUNK * _NBUF) == 0
  seq_per_w = b // _NW
  steps = seq_per_w // _SEQ_PER_CHUNK
  mesh = plsc.VectorSubcoreMesh(core_axis_name="c", subcore_axis_name="s")
  wstarts = _window_starts(l)

  @functools.partial(
      pl.kernel,
      mesh=mesh,
      out_type=jax.ShapeDtypeStruct((b, l, d), jnp.float32),
      scratch_types=(
          [pltpu.VMEM((seq_per_w, l), jnp.int32)]
          + [pltpu.VMEM((_SEQ_PER_CHUNK, l, d), jnp.float32)] * _NBUF
          + [pltpu.SemaphoreType.DMA] * (2 * _NBUF)
      ),
      compiler_params=pltpu.CompilerParams(use_tc_tiling_on_sc=False),
  )
  def gather_kernel(table_hbm, idx_hbm, out_hbm, idx_all, *bufs_and_sems):
    rows = bufs_and_sems[:_NBUF]
    gsem = bufs_and_sems[_NBUF:2 * _NBUF]
    osem = bufs_and_sems[2 * _NBUF:]
    wid = lax.axis_index("s") * _NUM_CORES + lax.axis_index("c")
    seq_base = wid * seq_per_w

    pltpu.sync_copy(idx_hbm.at[pl.ds(seq_base, seq_per_w)], idx_all)

    def fire_gathers(s, slot):
      for j in range(_SEQ_PER_CHUNK):
        pltpu.async_copy(
            table_hbm.at[idx_all.at[s * _SEQ_PER_CHUNK + j]],
            rows[slot].at[j],
            gsem[slot],
        )

    def drain_gathers(slot):
      for j in range(_SEQ_PER_CHUNK):
        pltpu.make_async_copy(
            table_hbm.at[idx_all.at[j]],
            rows[slot].at[j],
            gsem[slot],
        ).wait()

    for s0 in range(_LOOKAHEAD):
      fire_gathers(s0, s0)

    izeros = jnp.zeros((_LANES,), jnp.int32)
    ones = jnp.ones((_LANES,), jnp.int32)
    lane = lax.iota(jnp.int32, _LANES)

    def outer(t0, carry):
      for slot in range(_NBUF):
        s = t0 * _NBUF + slot
        drain_gathers(slot)

        acc = izeros
        for j in range(_SEQ_PER_CHUNK):
          for w in wstarts:
            idx16 = idx_all[s * _SEQ_PER_CHUNK + j, pl.ds(w, _LANES)]
            acc = acc | jnp.where(idx16 == _PAD_ID, ones, izeros)
        red = acc
        for shift in (8, 4, 2, 1):
          perm = (lane + shift) % _LANES
          red = red | jnp.take_along_axis(
              red, perm, axis=0, mode="promise_in_bounds")

        @pl.when(red[0] > 0)
        def _zero_rows():
          def zero_seq(j, c2):
            seq = s * _SEQ_PER_CHUNK + j
            for w in wstarts:
              idx16 = idx_all[seq, pl.ds(w, _LANES)]
              mf = jnp.where(idx16 == _PAD_ID, 0.0, 1.0)
              for r in range(_LANES):
                sc = mf[r]
                for c in range(d // _LANES):
                  vec = rows[slot][j, w + r, pl.ds(c * _LANES, _LANES)]
                  rows[slot][j, w + r, pl.ds(c * _LANES, _LANES)] = vec * sc
            return c2
          lax.fori_loop(0, _SEQ_PER_CHUNK, zero_seq, 0)

        pltpu.async_copy(
            rows[slot],
            out_hbm.at[pl.ds(seq_base + s * _SEQ_PER_CHUNK, _SEQ_PER_CHUNK)],
            osem[slot],
        )

        nslot = (slot + _LOOKAHEAD) % _NBUF
        sn = s + _LOOKAHEAD

        @pl.when(sn < steps)
        def _fire_ahead():
          @pl.when(s >= _LOOKAHEAD)
          def _drain_store():
            pltpu.make_async_copy(
                rows[nslot],
                out_hbm.at[pl.ds(seq_base, _SEQ_PER_CHUNK)],
                osem[nslot],
            ).wait()
          fire_gathers(sn, nslot)

      return carry

    lax.fori_loop(0, steps // _NBUF, outer, 0)

    for slot in range(_NBUF):
      pltpu.make_async_copy(
          rows[slot],
          out_hbm.at[pl.ds(seq_base, _SEQ_PER_CHUNK)],
          osem[slot],
      ).wait()

  return gather_kernel


def kernel(indices, table):
  b, l = indices.shape
  v, d = table.shape
  idx = indices.astype(jnp.int32)
  return _make_gather(b, l, d)(table, idx)


# speedup vs baseline: 1.3433x; 1.3433x over previous
import functools

import jax
import jax.numpy as jnp
from jax import lax
from jax.experimental import pallas as pl
from jax.experimental.pallas import tpu as pltpu
from jax.experimental.pallas import tpu_sc as plsc

_LANES = 16
_NUM_CORES = 2
_NUM_SUBCORES = 16
_NW = _NUM_CORES * _NUM_SUBCORES
_SEQ_PER_CHUNK = 4
_NBUF = 4
_LOOKAHEAD = 2
_PAD_ID = 0


def _window_starts(l: int):
  starts = list(range(0, l - _LANES + 1, _LANES))
  if starts[-1] + _LANES < l:
    starts.append(l - _LANES)
  return tuple(starts)


def _pad8(n: int) -> int:
  return (n + 7) // 8 * 8


def _make_gather(b: int, l: int, d: int):
  assert b % (_NW * _SEQ_PER_CHUNK * _NBUF) == 0
  seq_per_w = b // _NW
  steps = seq_per_w // _SEQ_PER_CHUNK
  lp, dp = _pad8(l), 128
  mesh = plsc.VectorSubcoreMesh(core_axis_name="c", subcore_axis_name="s")
  wstarts = _window_starts(l)

  @functools.partial(
      pl.kernel,
      mesh=mesh,
      out_type=jax.ShapeDtypeStruct((b, lp, dp), jnp.float32),
      scratch_types=(
          [pltpu.VMEM((seq_per_w, l), jnp.int32)]
          + [pltpu.VMEM((_SEQ_PER_CHUNK, l, d), jnp.float32)] * _NBUF
          + [pltpu.SemaphoreType.DMA] * (2 * _NBUF)
      ),
      compiler_params=pltpu.CompilerParams(use_tc_tiling_on_sc=False),
  )
  def gather_kernel(table_hbm, idx_hbm, out_hbm, idx_all, *bufs_and_sems):
    rows = bufs_and_sems[:_NBUF]
    gsem = bufs_and_sems[_NBUF:2 * _NBUF]
    osem = bufs_and_sems[2 * _NBUF:]
    wid = lax.axis_index("s") * _NUM_CORES + lax.axis_index("c")
    seq_base = wid * seq_per_w

    pltpu.sync_copy(idx_hbm.at[pl.ds(seq_base, seq_per_w)], idx_all)

    def out_slice(s):
      return out_hbm.at[
          pl.ds(seq_base + s * _SEQ_PER_CHUNK, _SEQ_PER_CHUNK),
          pl.ds(0, l),
          pl.ds(0, d),
      ]

    def fire_gathers(s, slot):
      for j in range(_SEQ_PER_CHUNK):
        pltpu.async_copy(
            table_hbm.at[idx_all.at[s * _SEQ_PER_CHUNK + j]],
            rows[slot].at[j],
            gsem[slot],
        )

    def drain_gathers(slot):
      for j in range(_SEQ_PER_CHUNK):
        pltpu.make_async_copy(
            table_hbm.at[idx_all.at[j]],
            rows[slot].at[j],
            gsem[slot],
        ).wait()

    for s0 in range(_LOOKAHEAD):
      fire_gathers(s0, s0)

    izeros = jnp.zeros((_LANES,), jnp.int32)
    ones = jnp.ones((_LANES,), jnp.int32)
    lane = lax.iota(jnp.int32, _LANES)

    def outer(t0, carry):
      for slot in range(_NBUF):
        s = t0 * _NBUF + slot
        drain_gathers(slot)

        acc = izeros
        for j in range(_SEQ_PER_CHUNK):
          for w in wstarts:
            idx16 = idx_all[s * _SEQ_PER_CHUNK + j, pl.ds(w, _LANES)]
            acc = acc | jnp.where(idx16 == _PAD_ID, ones, izeros)
        red = acc
        for shift in (8, 4, 2, 1):
          perm = (lane + shift) % _LANES
          red = red | jnp.take_along_axis(
              red, perm, axis=0, mode="promise_in_bounds")

        @pl.when(red[0] > 0)
        def _zero_rows():
          def zero_seq(j, c2):
            seq = s * _SEQ_PER_CHUNK + j
            for w in wstarts:
              idx16 = idx_all[seq, pl.ds(w, _LANES)]
              mf = jnp.where(idx16 == _PAD_ID, 0.0, 1.0)
              for r in range(_LANES):
                sc = mf[r]
                for c in range(d // _LANES):
                  vec = rows[slot][j, w + r, pl.ds(c * _LANES, _LANES)]
                  rows[slot][j, w + r, pl.ds(c * _LANES, _LANES)] = vec * sc
            return c2
          lax.fori_loop(0, _SEQ_PER_CHUNK, zero_seq, 0)

        pltpu.async_copy(rows[slot], out_slice(s), osem[slot])

        nslot = (slot + _LOOKAHEAD) % _NBUF
        sn = s + _LOOKAHEAD

        @pl.when(sn < steps)
        def _fire_ahead():
          @pl.when(s >= _LOOKAHEAD)
          def _drain_store():
            pltpu.make_async_copy(
                rows[nslot], out_slice(0), osem[nslot]).wait()
          fire_gathers(sn, nslot)

      return carry

    lax.fori_loop(0, steps // _NBUF, outer, 0)

    for slot in range(_NBUF):
      pltpu.make_async_copy(rows[slot], out_slice(0), osem[slot]).wait()

  return gather_kernel


def kernel(indices, table):
  b, l = indices.shape
  v, d = table.shape
  idx = indices.astype(jnp.int32)
  padded = _make_gather(b, l, d)(table, idx)
  return lax.slice(padded, (0, 0, 0), (b, l, d))
